# baseline (speedup 1.0000x reference)
import functools

import jax
import jax.numpy as jnp
from jax import lax
from jax.experimental import pallas as pl
from jax.experimental.pallas import tpu as pltpu
from jax.experimental.pallas import tpu_sc as plsc

B = 16384
D = 64
NEG = 20
VOCAB = 1000000
NC, NS, L = 2, 16, 16
NW = NC * NS
BPW = B // NW
PITCH = 128
NBLK = VOCAB // PITCH
TRIPS = NBLK // NW + 1
TAIL0 = VOCAB - PITCH

_MESH = plsc.VectorSubcoreMesh(
    core_axis_name="c", subcore_axis_name="s", num_cores=NC, num_subcores=NS
)


def _sc_relayout(t_hbm_arg, tail_arg):
    @functools.partial(
        pl.kernel,
        out_type=jax.ShapeDtypeStruct((VOCAB // 2, PITCH), jnp.float32),
        mesh=_MESH,
        compiler_params=pltpu.CompilerParams(
            use_tc_tiling_on_sc=True, needs_layout_passes=False),
        scratch_types=[
            pltpu.VMEM((2, D, PITCH), jnp.float32),
            pltpu.VMEM((2, D, PITCH), jnp.float32),
            pltpu.SemaphoreType.DMA,
            pltpu.SemaphoreType.DMA,
        ],
    )
    def k(t_hbm, tail_hbm, out_hbm, blk, obuf, gsem, wsem):
        wid = lax.axis_index("s") * NC + lax.axis_index("c")
        iota = lax.broadcasted_iota(jnp.int32, (L,), 0)
        dvs = [iota + q * L for q in range(D // L)]

        def transpose_block(s):
            bl = blk.at[s]
            ob = obuf.at[s]

            @plsc.parallel_loop(0, PITCH, unroll=16)
            def _(y):
                yv = jnp.full((L,), y, jnp.int32)
                orow = y // 2
                half = (y % 2) * D
                for q in range(D // L):
                    vals = plsc.load_gather(bl, [dvs[q], yv])
                    ob[orow, pl.ds(half + q * L, L)] = vals

        @pl.when(wid == 0)
        def _():
            pltpu.sync_copy(tail_hbm, blk.at[0])
            transpose_block(0)
            pltpu.sync_copy(obuf.at[0], out_hbm.at[pl.ds(TAIL0 // 2, D)])

        def fire(c, s):
            pltpu.async_copy(
                t_hbm.at[:, pl.ds(c * PITCH, PITCH)], blk.at[s], gsem)

        def wait_gather(s):
            pltpu.make_async_copy(
                t_hbm.at[:, pl.ds(0, PITCH)], blk.at[s], gsem).wait()

        def fire_wb(c, s):
            pltpu.async_copy(
                obuf.at[s], out_hbm.at[pl.ds(c * D, D)], wsem)

        def wait_wb():
            pltpu.make_async_copy(
                obuf.at[0], out_hbm.at[pl.ds(0, D)], wsem).wait()

        fire(wid, 0)

        def trip(j, carry):
            c = wid + j * NW
            s = j % 2
            valid = c < NBLK
            nc = c + NW

            @pl.when(valid)
            def _():
                wait_gather(s)

            @pl.when(nc < NBLK)
            def _():
                fire(nc, 1 - s)

            @pl.when(jnp.logical_and(valid, j >= 2))
            def _():
                wait_wb()

            @pl.when(valid)
            def _():
                transpose_block(s)
                fire_wb(c, s)

            return carry

        lax.fori_loop(0, TRIPS, trip, 0)
        wait_wb()
        wait_wb()

    return k(t_hbm_arg, tail_arg)


CB = 32
NCHUNK = BPW // CB
NROWS = CB * NEG
NIW = 128
NNI = NROWS // NIW
POS_SZ = B * L


def _sc_partial_dots(cent_idx, pos_idx, neg_idx, U, V):
    @functools.partial(
        pl.kernel,
        out_type=jax.ShapeDtypeStruct((B * (NEG + 1) * L,), jnp.float32),
        mesh=_MESH,
        compiler_params=pltpu.CompilerParams(use_tc_tiling_on_sc=False),
        scratch_types=[
            pltpu.VMEM((NCHUNK, CB), jnp.int32),
            pltpu.VMEM((NCHUNK, CB), jnp.int32),
            pltpu.VMEM((NCHUNK * NNI, NIW), jnp.int32),
            pltpu.VMEM((2, CB, D), jnp.float32),
            pltpu.VMEM((2, CB, D), jnp.float32),
            pltpu.VMEM((2, NROWS, D), jnp.float32),
            pltpu.VMEM((2, CB * L), jnp.float32),
            pltpu.VMEM((2, NROWS * L), jnp.float32),
            pltpu.SemaphoreType.DMA,
            pltpu.SemaphoreType.DMA,
            pltpu.SemaphoreType.DMA,
        ],
    )
    def k(cent_hbm, pos_hbm, neg_hbm, u_hbm, v_hbm, out_hbm, idx_c, idx_p,
          idx_n, cent_rows, pos_rows, neg_rows, part_p, part_n, sem0, sem1,
          wsem):
        wid = lax.axis_index("s") * NC + lax.axis_index("c")
        pltpu.sync_copy(cent_hbm.at[pl.ds(wid * NCHUNK, NCHUNK)], idx_c)
        pltpu.sync_copy(pos_hbm.at[pl.ds(wid * NCHUNK, NCHUNK)], idx_p)
        pltpu.sync_copy(neg_hbm.at[pl.ds(wid * NCHUNK * NNI, NCHUNK * NNI)], idx_n)
        sems = (sem0, sem1)

        def fire(i, s):
            sem = sems[s]
            copies = [
                pltpu.async_copy(
                    v_hbm.at[idx_n.at[i * NNI + j]],
                    neg_rows.at[s].at[pl.ds(j * NIW, NIW)],
                    sem,
                )
                for j in range(NNI)
            ]
            copies.append(
                pltpu.async_copy(v_hbm.at[idx_p.at[i]], pos_rows.at[s], sem))
            copies.append(
                pltpu.async_copy(u_hbm.at[idx_c.at[i]], cent_rows.at[s], sem))
            return copies

        pending = fire(0, 0)
        wb = []
        for i in range(NCHUNK):
            s = i % 2
            chunk = wid * NCHUNK + i
            for c in pending:
                c.wait()
            if i + 1 < NCHUNK:
                pending = fire(i + 1, (i + 1) % 2)
            for c in wb:
                c.wait()
            wb = []
            cr, pr, nr = cent_rows.at[s], pos_rows.at[s], neg_rows.at[s]
            pp_, pn_ = part_p.at[s], part_n.at[s]

            def unpack_row(ref, j):
                return (ref[j, pl.ds(0 * L, L)], ref[j, pl.ds(1 * L, L)],
                        ref[j, pl.ds(2 * L, L)], ref[j, pl.ds(3 * L, L)])

            def b_body(b, carry2, cr=cr, pr=pr, nr=nr, pp_=pp_, pn_=pn_):
                c0, c1, c2, c3 = unpack_row(cr, b)
                p0, p1, p2, p3 = unpack_row(pr, b)
                acc = p0 * c0 + p1 * c1 + p2 * c2 + p3 * c3
                pp_[pl.ds(b * L, L)] = acc
                for r in range(NEG):
                    row = b * NEG + r
                    n0, n1, n2, n3 = unpack_row(nr, row)
                    acc = n0 * c0 + n1 * c1 + n2 * c2 + n3 * c3
                    pn_[pl.ds(row * L, L)] = acc
                return carry2

            lax.fori_loop(0, CB, b_body, 0)
            wb = [
                pltpu.async_copy(
                    pp_, out_hbm.at[pl.ds(chunk * CB * L, CB * L)], wsem),
                pltpu.async_copy(
                    pn_,
                    out_hbm.at[pl.ds(POS_SZ + chunk * NROWS * L, NROWS * L)],
                    wsem,
                ),
            ]
        for c in wb:
            c.wait()

    return k(cent_idx, pos_idx, neg_idx, U, V)


_TC_GRID = 8
_TC_ROWS = B * (NEG + 1) * L // 128 // _TC_GRID
_POS_ROWS = POS_SZ // 128


def _tc_loss_body(pp_ref, out_ref):
    i = pl.program_id(0)
    x = pp_ref[...]
    lane = lax.broadcasted_iota(jnp.int32, (128, 8), 0)
    grp = lax.broadcasted_iota(jnp.int32, (128, 8), 1)
    onehot = (lane // L == grp).astype(jnp.float32)
    logits = jnp.dot(x, onehot, preferred_element_type=jnp.float32)
    row = i * _TC_ROWS + lax.broadcasted_iota(jnp.int32, (_TC_ROWS, 8), 0)
    sign = jnp.where(row < _POS_ROWS, 1.0, -1.0)
    z = sign * logits
    contrib = jnp.minimum(z, 0.0) - jnp.log1p(jnp.exp(-jnp.abs(z)))

    @pl.when(i == 0)
    def _():
        out_ref[...] = jnp.zeros_like(out_ref)

    out_ref[...] = out_ref[...] - jnp.sum(contrib)


def _tc_loss(pp):
    pp2 = pp.reshape(B * (NEG + 1) * L // 128, 128)
    return pl.pallas_call(
        _tc_loss_body,
        out_shape=jax.ShapeDtypeStruct((1, 1), jnp.float32),
        grid=(_TC_GRID,),
        in_specs=[pl.BlockSpec((_TC_ROWS, 128), lambda i: (i, 0))],
        out_specs=pl.BlockSpec((1, 1), lambda i: (0, 0)),
    )(pp2)


def kernel(cent_word, pos_word, neg_word, U, V):
    cent_idx = cent_word.astype(jnp.int32).reshape(B // CB, CB)
    pos_idx = pos_word.astype(jnp.int32).reshape(B // CB, CB)
    neg_idx = neg_word.astype(jnp.int32).reshape(B * NEG // NIW, NIW)
    U2 = _sc_relayout(U.T, U[TAIL0:, :].T).reshape(VOCAB, D)
    V2 = _sc_relayout(V.T, V[TAIL0:, :].T).reshape(VOCAB, D)
    pp = _sc_partial_dots(cent_idx, pos_idx, neg_idx, U2, V2)
    return _tc_loss(pp).reshape(())


# speedup vs baseline: 1.4173x; 1.4173x over previous
import functools

import jax
import jax.numpy as jnp
from jax import lax
from jax.experimental import pallas as pl
from jax.experimental.pallas import tpu as pltpu
from jax.experimental.pallas import tpu_sc as plsc

B = 16384
D = 64
NEG = 20
NC, NS, L = 2, 16, 16
NW = NC * NS
BPW = B // NW
CB = 32
NCHUNK = BPW // CB
NROWS = CB * NEG
NIW = 128
NNI = NROWS // NIW
POS_SZ = B * L


def _sc_partial_dots(cent_idx, pos_idx, neg_idx, U, V):
    mesh = plsc.VectorSubcoreMesh(
        core_axis_name="c", subcore_axis_name="s", num_cores=NC, num_subcores=NS
    )

    @functools.partial(
        pl.kernel,
        out_type=jax.ShapeDtypeStruct((B * (NEG + 1) * L,), jnp.float32),
        mesh=mesh,
        compiler_params=pltpu.CompilerParams(use_tc_tiling_on_sc=False),
        scratch_types=[
            pltpu.VMEM((NCHUNK, CB), jnp.int32),
            pltpu.VMEM((NCHUNK, CB), jnp.int32),
            pltpu.VMEM((NCHUNK * NNI, NIW), jnp.int32),
            pltpu.VMEM((2, CB, D), jnp.float32),
            pltpu.VMEM((2, CB, D), jnp.float32),
            pltpu.VMEM((2, NROWS, D), jnp.float32),
            pltpu.VMEM((2, CB * L), jnp.float32),
            pltpu.VMEM((2, NROWS * L), jnp.float32),
            pltpu.SemaphoreType.DMA,
            pltpu.SemaphoreType.DMA,
            pltpu.SemaphoreType.DMA,
        ],
    )
    def k(cent_hbm, pos_hbm, neg_hbm, u_hbm, v_hbm, out_hbm, idx_c, idx_p,
          idx_n, cent_rows, pos_rows, neg_rows, part_p, part_n, sem0, sem1,
          wsem):
        wid = lax.axis_index("s") * NC + lax.axis_index("c")
        pltpu.sync_copy(cent_hbm.at[pl.ds(wid * NCHUNK, NCHUNK)], idx_c)
        pltpu.sync_copy(pos_hbm.at[pl.ds(wid * NCHUNK, NCHUNK)], idx_p)
        pltpu.sync_copy(neg_hbm.at[pl.ds(wid * NCHUNK * NNI, NCHUNK * NNI)], idx_n)
        sems = (sem0, sem1)

        def fire(i, s):
            sem = sems[s]
            copies = [
                pltpu.async_copy(
                    v_hbm.at[idx_n.at[i * NNI + j]],
                    neg_rows.at[s].at[pl.ds(j * NIW, NIW)],
                    sem,
                )
                for j in range(NNI)
            ]
            copies.append(
                pltpu.async_copy(v_hbm.at[idx_p.at[i]], pos_rows.at[s], sem))
            copies.append(
                pltpu.async_copy(u_hbm.at[idx_c.at[i]], cent_rows.at[s], sem))
            return copies

        pending = fire(0, 0)
        wb = []
        for i in range(NCHUNK):
            s = i % 2
            chunk = wid * NCHUNK + i
            for c in pending:
                c.wait()
            if i + 1 < NCHUNK:
                pending = fire(i + 1, (i + 1) % 2)
            for c in wb:
                c.wait()
            wb = []
            cr, pr, nr = cent_rows.at[s], pos_rows.at[s], neg_rows.at[s]
            pp_, pn_ = part_p.at[s], part_n.at[s]

            def unpack_row(ref, j):
                return (ref[j, pl.ds(0 * L, L)], ref[j, pl.ds(1 * L, L)],
                        ref[j, pl.ds(2 * L, L)], ref[j, pl.ds(3 * L, L)])

            def b_body(b, carry2, cr=cr, pr=pr, nr=nr, pp_=pp_, pn_=pn_):
                c0, c1, c2, c3 = unpack_row(cr, b)
                p0, p1, p2, p3 = unpack_row(pr, b)
                acc = p0 * c0 + p1 * c1 + p2 * c2 + p3 * c3
                pp_[pl.ds(b * L, L)] = acc
                for r in range(NEG):
                    row = b * NEG + r
                    n0, n1, n2, n3 = unpack_row(nr, row)
                    acc = n0 * c0 + n1 * c1 + n2 * c2 + n3 * c3
                    pn_[pl.ds(row * L, L)] = acc
                return carry2

            lax.fori_loop(0, CB, b_body, 0)
            wb = [
                pltpu.async_copy(
                    pp_, out_hbm.at[pl.ds(chunk * CB * L, CB * L)], wsem),
                pltpu.async_copy(
                    pn_,
                    out_hbm.at[pl.ds(POS_SZ + chunk * NROWS * L, NROWS * L)],
                    wsem,
                ),
            ]
        for c in wb:
            c.wait()

    return k(cent_idx, pos_idx, neg_idx, U, V)


_TC_GRID = 8
_TC_ROWS = B * (NEG + 1) * L // 128 // _TC_GRID
_POS_ROWS = POS_SZ // 128


def _tc_loss_body(pp_ref, out_ref):
    i = pl.program_id(0)
    x = pp_ref[...]
    lane = lax.broadcasted_iota(jnp.int32, (128, 8), 0)
    grp = lax.broadcasted_iota(jnp.int32, (128, 8), 1)
    onehot = (lane // L == grp).astype(jnp.float32)
    logits = jnp.dot(x, onehot, preferred_element_type=jnp.float32)
    row = i * _TC_ROWS + lax.broadcasted_iota(jnp.int32, (_TC_ROWS, 8), 0)
    sign = jnp.where(row < _POS_ROWS, 1.0, -1.0)
    z = sign * logits
    contrib = jnp.minimum(z, 0.0) - jnp.log1p(jnp.exp(-jnp.abs(z)))

    @pl.when(i == 0)
    def _():
        out_ref[...] = jnp.zeros_like(out_ref)

    out_ref[...] = out_ref[...] - jnp.sum(contrib)


def _tc_loss(pp):
    pp2 = pp.reshape(B * (NEG + 1) * L // 128, 128)
    return pl.pallas_call(
        _tc_loss_body,
        out_shape=jax.ShapeDtypeStruct((1, 1), jnp.float32),
        grid=(_TC_GRID,),
        in_specs=[pl.BlockSpec((_TC_ROWS, 128), lambda i: (i, 0))],
        out_specs=pl.BlockSpec((1, 1), lambda i: (0, 0)),
    )(pp2)


def kernel(cent_word, pos_word, neg_word, U, V):
    cent_idx = cent_word.astype(jnp.int32).reshape(B // CB, CB)
    pos_idx = pos_word.astype(jnp.int32).reshape(B // CB, CB)
    neg_idx = neg_word.astype(jnp.int32).reshape(B * NEG // NIW, NIW)
    pp = _sc_partial_dots(cent_idx, pos_idx, neg_idx, U, V)
    return _tc_loss(pp).reshape(())
